# baseline (speedup 1.0000x reference)
import functools

import jax
import jax.numpy as jnp
from jax import lax
from jax.experimental import pallas as pl
from jax.experimental.pallas import tpu as pltpu
from jax.experimental.pallas import tpu_sc as plsc

H = 256
F32 = jnp.float32

_NCORES = 2
_NSUB = 16
_NW = _NCORES * _NSUB


def _dot(a, b):
    return jax.lax.dot_general(a, b, (((1,), (0,)), ((), ())),
                               preferred_element_type=F32)


def _leaky(x):
    return jnp.where(x >= 0, x, 0.2 * x)


def _node_pre_body(nfgc, nfgn, nfgc1, nfgn1,
                   wc, bc, wn, bn,
                   wca, bca, wcb, bcb, wna, bna, wnb, bnb,
                   xgc1_o, xgn1_o, xgc_o, xgn_o):
    xgc1_o[...] = _dot(nfgc1[...], wc[...]) + bc[...]
    xgn1_o[...] = _dot(nfgn1[...], wn[...]) + bn[...]
    x = nfgc[...]
    xgc_o[...] = _dot(_dot(x, wca[...]) + bca[...], wcb[...]) + bcb[...] + x
    x = nfgn[...]
    xgn_o[...] = _dot(_dot(x, wna[...]) + bna[...], wnb[...]) + bnb[...] + x


def _node_pre(nfgc, nfgn, nfgc1, nfgn1, ws, N):
    B = 1000
    row = pl.BlockSpec((B, H), lambda i: (i, 0))
    wsp = pl.BlockSpec((H, H), lambda i: (0, 0))
    bsp = pl.BlockSpec((1, H), lambda i: (0, 0))
    n_w = len(ws)
    return pl.pallas_call(
        _node_pre_body,
        grid=(N // B,),
        in_specs=[row] * 4 + [wsp, bsp] * (n_w // 2),
        out_specs=[row] * 4,
        out_shape=[jax.ShapeDtypeStruct((N, H), F32)] * 4,
    )(nfgc, nfgn, nfgc1, nfgn1, *ws)


@functools.lru_cache(maxsize=None)
def _make_gather(E):
    epw = E // _NW
    C = 200
    nch = epw // C
    mesh = plsc.VectorSubcoreMesh(core_axis_name="c", subcore_axis_name="s")

    @functools.partial(
        pl.kernel, mesh=mesh,
        out_type=(jax.ShapeDtypeStruct((E, H), F32),
                  jax.ShapeDtypeStruct((E, H), F32)),
        scratch_types=[
            pltpu.VMEM((epw,), jnp.int32),
            pltpu.VMEM((epw,), jnp.int32),
            pltpu.VMEM((C, H), F32),
            pltpu.VMEM((C, H), F32),
            pltpu.SemaphoreType.DMA,
            pltpu.SemaphoreType.DMA,
        ],
    )
    def k(xsrc, xdst, sidx, didx, osrc, odst, siv, div, bufs, bufd, sem1, sem2):
        wid = lax.axis_index("s") * _NCORES + lax.axis_index("c")
        base = wid * epw
        pltpu.sync_copy(sidx.at[pl.ds(base, epw)], siv)
        pltpu.sync_copy(didx.at[pl.ds(base, epw)], div)

        def body(i, carry):
            off = i * C
            cs = pltpu.async_copy(xsrc.at[siv.at[pl.ds(off, C)]], bufs, sem1)
            cd = pltpu.async_copy(xdst.at[div.at[pl.ds(off, C)]], bufd, sem2)
            cs.wait()
            pltpu.sync_copy(bufs, osrc.at[pl.ds(base + off, C)])
            cd.wait()
            pltpu.sync_copy(bufd, odst.at[pl.ds(base + off, C)])
            return carry

        lax.fori_loop(0, nch, body, 0)

    return k


def _dot_t(a, b):
    return jax.lax.dot_general(a, b, (((0,), (0,)), ((), ())),
                               preferred_element_type=F32)


def _edge_mlp_body(src, dst, w1s, w1d, b1, wk2, bk, w2a, b2a, w2b, b2bc,
                   f1_o, f2t_o):
    h = _dot(src[...], w1s[...]) + _dot(dst[...], w1d[...]) + b1[...]
    h = _leaky(h)
    kv = jax.nn.sigmoid(_dot(h, wk2[...]) + bk[...])
    f1_o[...] = (_dot(h, w2a[...]) + b2a[...]) * kv
    kv_row = jax.nn.sigmoid(
        jax.lax.dot_general(wk2[...], h, (((0,), (1,)), ((), ())))
        + bk[...])
    f2t_o[...] = (jax.lax.dot_general(w2b[...], h, (((1,), (1,)), ((), ())))
                  + b2bc[...]) * kv_row


def _edge_mlp(src_g, dst_g, mp, E):
    B = 1280
    W1, b1 = mp['l1']
    W2, b2 = mp['l2']
    w1s = W1[:, :H].T
    w1d = W1[:, H:].T
    wk2 = W2[0].reshape(2 * H, 1)
    bk = b2[0].reshape(1, 1)
    w2a = W2[1:1 + H].T
    b2a = b2[1:1 + H].reshape(1, H)
    w2b = W2[1 + H:]
    b2bc = b2[1 + H:].reshape(H, 1)

    row = pl.BlockSpec((B, H), lambda i: (i, 0))
    colt = pl.BlockSpec((H, B), lambda i: (0, i))
    c = lambda shape: pl.BlockSpec(shape, lambda i: (0,) * len(shape))
    return pl.pallas_call(
        _edge_mlp_body,
        grid=(E // B,),
        in_specs=[row, row,
                  c((H, 2 * H)), c((H, 2 * H)), c((1, 2 * H)),
                  c((2 * H, 1)), c((1, 1)),
                  c((2 * H, H)), c((1, H)), c((H, 2 * H)), c((H, 1))],
        out_specs=[row, colt],
        out_shape=[jax.ShapeDtypeStruct((E, H), F32),
                   jax.ShapeDtypeStruct((H, E), F32)],
    )(src_g, dst_g, w1s, w1d, b1.reshape(1, 2 * H), wk2, bk,
      w2a, b2a, w2b, b2bc)


@functools.lru_cache(maxsize=None)
def _make_segsum(E, N):
    eps = E // _NSUB
    C = 200
    nch = eps // C
    NPAD = 10240
    ZR = NPAD // _NSUB
    RO = (N // _NSUB) // 8 * 8
    HH = H // 2
    mesh = plsc.VectorSubcoreMesh(core_axis_name="c", subcore_axis_name="s")

    @functools.partial(
        pl.kernel, mesh=mesh,
        out_type=jax.ShapeDtypeStruct((N, H), F32),
        scratch_types=[
            pltpu.VMEM_SHARED((NPAD, HH), F32),
            pltpu.VMEM((128, HH), F32),
            pltpu.VMEM((C,), jnp.int32),
            pltpu.VMEM((C, HH), F32),
        ],
    )
    def k(vals, didx, out, acc, zbuf, idxv, buf):
        ci = lax.axis_index("c")
        s = lax.axis_index("s")

        def zrow(i, carry):
            for j in range(HH // 16):
                zbuf[i, pl.ds(j * 16, 16)] = jnp.zeros((16,), F32)
            return carry
        lax.fori_loop(0, 128, zrow, 0)

        for t in range(ZR // 128):
            pltpu.sync_copy(zbuf, acc.at[pl.ds(s * ZR + t * 128, 128)])
        plsc.subcore_barrier()

        def body(i, carry):
            off = s * eps + i * C
            pltpu.sync_copy(didx.at[pl.ds(off, C)], idxv)
            pltpu.sync_copy(vals.at[pl.ds(off, C), pl.ds(ci * HH, HH)], buf)
            pltpu.sync_copy(buf, acc.at[idxv], add=True)
            return carry
        lax.fori_loop(0, nch, body, 0)

        plsc.subcore_barrier()
        pltpu.sync_copy(acc.at[pl.ds(s * RO, RO)],
                        out.at[pl.ds(s * RO, RO), pl.ds(ci * HH, HH)])

        @pl.when(s == 0)
        def _():
            pltpu.sync_copy(acc.at[pl.ds(_NSUB * RO, N - _NSUB * RO)],
                            out.at[pl.ds(_NSUB * RO, N - _NSUB * RO),
                                   pl.ds(ci * HH, HH)])

    return k


@functools.lru_cache(maxsize=None)
def _make_segmax(E, N, NPADT):
    NH = NPADT // 2
    C = 1280
    nch = E // C
    mesh = plsc.VectorSubcoreMesh(core_axis_name="c", subcore_axis_name="s")

    @functools.partial(
        pl.kernel, mesh=mesh,
        out_type=jax.ShapeDtypeStruct((H, NPADT), F32),
        compiler_params=pltpu.CompilerParams(needs_layout_passes=False),
        scratch_types=[
            pltpu.VMEM((16, NH), F32),
            pltpu.VMEM((C,), jnp.int32),
            pltpu.VMEM((16, C), F32),
        ],
    )
    def k(valst, didx, out, acc, idxv, buf):
        ci = lax.axis_index("c")
        s = lax.axis_index("s")
        lo = ci * NH
        l16 = lax.iota(jnp.int32, 16)

        neg = jnp.full((16,), -jnp.inf, F32)

        def init(i, carry):
            for r in range(16):
                acc[r, pl.ds(i * 16, 16)] = neg
            return carry
        lax.fori_loop(0, NH // 16, init, 0)

        def chunk(i, carry):
            off = i * C
            pltpu.sync_copy(didx.at[pl.ds(off, C)], idxv)
            pltpu.sync_copy(valst.at[pl.ds(s * 16, 16), pl.ds(off, C)], buf)

            def grp(j, c2):
                dl = idxv[pl.ds(j * 16, 16)] - lo
                for kk in range(16):
                    dk = dl[kk]

                    @pl.when((dk >= 0) & (dk < NH))
                    def _():
                        col = jnp.full((16,), dk, jnp.int32)
                        v = plsc.load_gather(
                            buf, [l16, jnp.full((16,), j * 16 + kk,
                                                jnp.int32)])
                        a = plsc.load_gather(acc, [l16, col])
                        plsc.store_scatter(acc, [l16, col],
                                           jnp.maximum(a, v))
                return c2
            lax.fori_loop(0, C // 16, grp, 0)
            return carry
        lax.fori_loop(0, nch, chunk, 0)

        pltpu.sync_copy(acc, out.at[pl.ds(s * 16, 16), pl.ds(lo, NH)])

    return k


def _fix_t_body(mt_in, m_out):
    mxt = mt_in[...]
    m_out[...] = jnp.where(mxt == -jnp.inf, 0.0, mxt).T


def _fix_t(mt, NPADT):
    B = 1280
    return pl.pallas_call(
        _fix_t_body,
        grid=(NPADT // B,),
        in_specs=[pl.BlockSpec((H, B), lambda i: (0, i))],
        out_specs=pl.BlockSpec((B, H), lambda i: (i, 0)),
        out_shape=jax.ShapeDtypeStruct((NPADT, H), F32),
    )(mt)


def _node_mid_body(xgn, ssum, smax, xgn1, xgc,
                   wr1, wr2, wr3, br, wg, bg, wp1, wp2, bp,
                   wna, bna, wnb, bnb, wca, bca, wcb, bcb,
                   xgn_o, xgc_o):
    x = xgn[...]
    nx = (_dot(x, wr1[...]) + _dot(ssum[...], wr2[...])
          + _dot(smax[...], wr3[...]) + br[...])
    nx = _dot(nx, wg[...]) + bg[...]
    x = x + _dot(nx, wp1[...]) + _dot(xgn1[...], wp2[...]) + bp[...]
    xgn_o[...] = _dot(_dot(x, wna[...]) + bna[...], wnb[...]) + bnb[...] + x
    x = xgc[...]
    xgc_o[...] = _dot(_dot(x, wca[...]) + bca[...], wcb[...]) + bcb[...] + x


def _node_mid(xgn, ssum, smax, xgn1, xgc, ws, N):
    B = 1000
    row = pl.BlockSpec((B, H), lambda i: (i, 0))
    specs = []
    for w in ws:
        specs.append(pl.BlockSpec(w.shape, lambda i: (0, 0)))
    return pl.pallas_call(
        _node_mid_body,
        grid=(N // B,),
        in_specs=[row] * 5 + specs,
        out_specs=[row, row],
        out_shape=[jax.ShapeDtypeStruct((N, H), F32)] * 2,
    )(xgn, ssum, smax, xgn1, xgc, *ws)


def _node_post_body(xgc, ssum, smax, xgc1,
                    wr1, wr2, wr3, br, wg, bg, wp1, wp2, bp,
                    xgc_o):
    x = xgc[...]
    nx = (_dot(x, wr1[...]) + _dot(ssum[...], wr2[...])
          + _dot(smax[...], wr3[...]) + br[...])
    nx = _dot(nx, wg[...]) + bg[...]
    xgc_o[...] = x + _dot(nx, wp1[...]) + _dot(xgc1[...], wp2[...]) + bp[...]


def _node_post(xgc, ssum, smax, xgc1, ws, N):
    B = 1000
    row = pl.BlockSpec((B, H), lambda i: (i, 0))
    specs = [pl.BlockSpec(w.shape, lambda i: (0, 0)) for w in ws]
    return pl.pallas_call(
        _node_post_body,
        grid=(N // B,),
        in_specs=[row] * 4 + specs,
        out_specs=row,
        out_shape=jax.ShapeDtypeStruct((N, H), F32),
    )(xgc, ssum, smax, xgc1, *ws)


def _lin_t(p):
    return p[0].T, p[1].reshape(1, -1)


def kernel(nf_gc, nf_gn, nf_gc_in1, nf_gn_in1, edge_c2n, edge_n2c, params):
    p = params
    NC = nf_gc.shape[0]
    NN = nf_gn.shape[0]
    E = edge_c2n.shape[1]

    wc, bc = _lin_t(p['gc_in1'])
    wn, bn = _lin_t(p['gn_in1'])
    wca, bca = _lin_t(p['res_gc_1']['l1'])
    wcb, bcb = _lin_t(p['res_gc_1']['l2'])
    wna, bna = _lin_t(p['res_gn_1']['l1'])
    wnb, bnb = _lin_t(p['res_gn_1']['l2'])
    xgc1, xgn1, xgc, xgn = _node_pre(
        nf_gc, nf_gn, nf_gc_in1, nf_gn_in1,
        (wc, bc, wn, bn, wca, bca, wcb, bcb, wna, bna, wnb, bnb), NC)

    NPADT = 10240
    src_g, dst_g = _make_gather(E)(xgc, xgn, edge_c2n[0], edge_c2n[1])
    f1, f2t = _edge_mlp(src_g, dst_g, p['msg_c2n'], E)
    s1 = _make_segsum(E, NN)(f1, edge_c2n[1])
    m1 = _fix_t(_make_segmax(E, NN, NPADT)(f2t, edge_c2n[1]), NPADT)

    Wr, br = p['red_c2n']
    wg, bg = _lin_t(p['Gcn'])
    Wp, bp = p['postCatGcn']
    w2na, b2na = _lin_t(p['res_gn_2']['l1'])
    w2nb, b2nb = _lin_t(p['res_gn_2']['l2'])
    w2ca, b2ca = _lin_t(p['res_gc_2']['l1'])
    w2cb, b2cb = _lin_t(p['res_gc_2']['l2'])
    mid_ws = (Wr[:, :H].T, Wr[:, H:2 * H].T, Wr[:, 2 * H:].T,
              br.reshape(1, H), wg, bg,
              Wp[:, :H].T, Wp[:, H:].T, bp.reshape(1, H),
              w2na, b2na, w2nb, b2nb, w2ca, b2ca, w2cb, b2cb)
    xgn2, xgc2 = _node_mid(xgn, s1, m1, xgn1, xgc, mid_ws, NN)

    src_g2, dst_g2 = _make_gather(E)(xgn2, xgc2, edge_n2c[0], edge_n2c[1])
    f1b, f2tb = _edge_mlp(src_g2, dst_g2, p['msg_n2c'], E)
    s2 = _make_segsum(E, NC)(f1b, edge_n2c[1])
    m2 = _fix_t(_make_segmax(E, NC, NPADT)(f2tb, edge_n2c[1]), NPADT)

    Wr2, br2 = p['red_n2c']
    wg2, bg2 = _lin_t(p['Gnc'])
    Wp2, bp2 = p['postCatGnc']
    post_ws = (Wr2[:, :H].T, Wr2[:, H:2 * H].T, Wr2[:, 2 * H:].T,
               br2.reshape(1, H), wg2, bg2,
               Wp2[:, :H].T, Wp2[:, H:].T, bp2.reshape(1, H))
    xgc_out = _node_post(xgc2, s2, m2, xgc1, post_ws, NC)

    return (xgc_out, xgn2)
